# speedup vs baseline: 1.1648x; 1.1648x over previous
import jax
import jax.numpy as jnp
from jax.experimental import pallas as pl
from jax.experimental.pallas import tpu as pltpu

_SCALE = 8.0 / 8.0

_TN = 512


def _qkv_lora_body(idx_ref, x_ref, wt_ref, ac_ref, bc_ref, bias_ref, o_ref):
    xb = x_ref[0]
    acc = jnp.dot(xb, wt_ref[...], preferred_element_type=jnp.float32)
    r = jnp.dot(xb, ac_ref[0], preferred_element_type=jnp.float32)
    upd = jnp.dot(r.astype(jnp.bfloat16), bc_ref[0],
                  preferred_element_type=jnp.float32)
    o_ref[0] = acc + upd + bias_ref[...]


def kernel(x, weight, bias, A_q_pool, B_q_pool, A_v_pool, B_v_pool, idx,
           frozen_mask):
    B, N, D = x.shape
    O = weight.shape[0]
    P, _, R = A_q_pool.shape

    xb = x.astype(jnp.bfloat16)
    wt = weight.T.astype(jnp.bfloat16)
    bias2 = bias.reshape(1, O)

    a_c = jnp.concatenate([A_q_pool, A_v_pool], axis=2).astype(jnp.bfloat16)
    b_c = jnp.zeros((P, 2 * R, O), jnp.float32)
    b_c = b_c.at[:, :R, :D].set(_SCALE * B_q_pool)
    b_c = b_c.at[:, R:, O - D:].set(_SCALE * B_v_pool)
    b_c = b_c.astype(jnp.bfloat16)

    idx32 = idx[:, 0].astype(jnp.int32)

    grid = (B, N // _TN)
    grid_spec = pltpu.PrefetchScalarGridSpec(
        num_scalar_prefetch=1,
        grid=grid,
        in_specs=[
            pl.BlockSpec((1, _TN, D), lambda b, n, idx_ref: (b, n, 0)),
            pl.BlockSpec((D, O), lambda b, n, idx_ref: (0, 0)),
            pl.BlockSpec((1, D, 2 * R), lambda b, n, idx_ref: (idx_ref[b], 0, 0)),
            pl.BlockSpec((1, 2 * R, O), lambda b, n, idx_ref: (idx_ref[b], 0, 0)),
            pl.BlockSpec((1, O), lambda b, n, idx_ref: (0, 0)),
        ],
        out_specs=pl.BlockSpec((1, _TN, O), lambda b, n, idx_ref: (b, n, 0)),
    )

    out = pl.pallas_call(
        _qkv_lora_body,
        out_shape=jax.ShapeDtypeStruct((B, N, O), jnp.float32),
        grid_spec=grid_spec,
        compiler_params=pltpu.CompilerParams(
            dimension_semantics=("parallel", "arbitrary"),
            vmem_limit_bytes=56 * 1024 * 1024,
        ),
        name="qkv_lora_fused",
    )(idx32, xb, wt, a_c, b_c, bias2)
    return out
